# baseline (speedup 1.0000x reference)
import functools

import jax
import jax.numpy as jnp
from jax import lax
from jax.experimental import pallas as pl
from jax.experimental.pallas import tpu as pltpu
from jax.experimental.pallas import tpu_sc as plsc

N = 10000
E = 320000
D = 128
G = 16
O = 64

NC, NS = 2, 16
NW = NC * NS
EPW = E // NW
CH = 80
EPAD = 10240
NCHUNK = EPAD // CH
IB = 32
NBLK = NCHUNK // IB
NPAD = 10240
RPT = NPAD // NS


def _fill(buf, rows, width, value):
    v = jnp.full((16,), value, jnp.float32)

    def row(r, carry):
        for cidx in range(width // 16):
            buf[r, pl.ds(cidx * 16, 16)] = v
        return carry

    lax.fori_loop(0, rows, row, 0)


def _zero_acc(buf, acc_sp, s):
    _fill(buf, CH, D, 0.0)
    for k in range(RPT // CH):
        pltpu.sync_copy(buf, acc_sp.at[pl.ds(s * RPT + k * CH, CH)])


def _sc_count_body(dst_hbm, out_hbm, dst_v, ones, acc_sp):
    c = lax.axis_index("c")
    s = lax.axis_index("s")
    wid = s * NC + c
    _zero_acc(ones, acc_sp, s)
    plsc.subcore_barrier()
    _fill(ones, CH, D, 1.0)

    def body(j, carry):
        pltpu.sync_copy(ones, acc_sp.at[dst_v.at[j]], add=True)
        return carry

    for blk in range(NBLK):
        pltpu.sync_copy(dst_hbm.at[wid, pl.ds(blk * IB, IB)], dst_v)
        lax.fori_loop(0, IB, body, 0)
    plsc.subcore_barrier()
    pltpu.sync_copy(acc_sp.at[pl.ds(s * RPT, RPT)],
                    out_hbm.at[c, pl.ds(s * RPT, RPT)])


def _sc_scatter_body(y_hbm, src_hbm, dst_hbm, out_hbm,
                     src_v, dst_v, buf0, buf1, acc_sp, sem0, sem1):
    c = lax.axis_index("c")
    s = lax.axis_index("s")
    wid = s * NC + c
    _zero_acc(buf0, acc_sp, s)
    plsc.subcore_barrier()

    def body(g, carry):
        j = 2 * g
        pltpu.make_async_copy(y_hbm.at[src_v.at[j]], buf0, sem0).wait()
        pltpu.sync_copy(buf0, acc_sp.at[dst_v.at[j]], add=True)
        pltpu.async_copy(y_hbm.at[src_v.at[j + 2]], buf0, sem0)
        pltpu.make_async_copy(y_hbm.at[src_v.at[j + 1]], buf1, sem1).wait()
        pltpu.sync_copy(buf1, acc_sp.at[dst_v.at[j + 1]], add=True)
        pltpu.async_copy(y_hbm.at[src_v.at[j + 3]], buf1, sem1)
        return carry

    for blk in range(NBLK):
        pltpu.sync_copy(src_hbm.at[wid, pl.ds(blk * IB, IB)], src_v)
        pltpu.sync_copy(dst_hbm.at[wid, pl.ds(blk * IB, IB)], dst_v)
        pltpu.async_copy(y_hbm.at[src_v.at[0]], buf0, sem0)
        pltpu.async_copy(y_hbm.at[src_v.at[1]], buf1, sem1)
        lax.fori_loop(0, IB // 2 - 1, body, 0)
        j = IB - 2
        pltpu.make_async_copy(y_hbm.at[src_v.at[j]], buf0, sem0).wait()
        pltpu.sync_copy(buf0, acc_sp.at[dst_v.at[j]], add=True)
        pltpu.make_async_copy(y_hbm.at[src_v.at[j + 1]], buf1, sem1).wait()
        pltpu.sync_copy(buf1, acc_sp.at[dst_v.at[j + 1]], add=True)
    plsc.subcore_barrier()
    pltpu.sync_copy(acc_sp.at[pl.ds(s * RPT, RPT)],
                    out_hbm.at[c, pl.ds(s * RPT, RPT)])


@functools.lru_cache(maxsize=None)
def _sc_kernels():
    mesh = plsc.VectorSubcoreMesh(core_axis_name="c", subcore_axis_name="s")
    count = pl.kernel(
        _sc_count_body,
        out_type=jax.ShapeDtypeStruct((NC, NPAD, D), jnp.float32),
        mesh=mesh,
        scratch_types=[
            pltpu.VMEM((IB, CH), jnp.int32),
            pltpu.VMEM((CH, D), jnp.float32),
            pltpu.VMEM_SHARED((NPAD, D), jnp.float32),
        ],
    )
    scatter = pl.kernel(
        _sc_scatter_body,
        out_type=jax.ShapeDtypeStruct((NC, NPAD, D), jnp.float32),
        mesh=mesh,
        scratch_types=[
            pltpu.VMEM((IB, CH), jnp.int32),
            pltpu.VMEM((IB, CH), jnp.int32),
            pltpu.VMEM((CH, D), jnp.float32),
            pltpu.VMEM((CH, D), jnp.float32),
            pltpu.VMEM_SHARED((NPAD, D), jnp.float32),
            pltpu.SemaphoreType.DMA,
            pltpu.SemaphoreType.DMA,
        ],
    )
    return count, scatter


RB = 1000
NRB = N // RB


def _dinv_from(cnt_blk):
    return lax.rsqrt(cnt_blk[0] + cnt_blk[1] + 1.0)


def _mm1_body(cnt_ref, x_ref, w_ref, y_ref):
    dinv = _dinv_from(cnt_ref[...])
    y_ref[...] = jnp.dot(x_ref[...], w_ref[...],
                         preferred_element_type=jnp.float32) * dinv


_mm1 = pl.pallas_call(
    _mm1_body,
    grid=(NRB,),
    in_specs=[
        pl.BlockSpec((NC, RB, D), lambda i: (0, i, 0)),
        pl.BlockSpec((RB, D), lambda i: (i, 0)),
        pl.BlockSpec((D, D), lambda i: (0, 0)),
    ],
    out_specs=pl.BlockSpec((RB, D), lambda i: (i, 0)),
    out_shape=jax.ShapeDtypeStruct((N, D), jnp.float32),
)


def _layer_body(cnt_ref, a_ref, y_ref, b_ref, w_ref, o_ref):
    dinv = _dinv_from(cnt_ref[...])
    ab = a_ref[...]
    h = jnp.maximum((ab[0] + ab[1] + y_ref[...]) * dinv + b_ref[...], 0.0)
    o_ref[...] = jnp.dot(h, w_ref[...],
                         preferred_element_type=jnp.float32) * dinv


_layer = pl.pallas_call(
    _layer_body,
    grid=(NRB,),
    in_specs=[
        pl.BlockSpec((NC, RB, D), lambda i: (0, i, 0)),
        pl.BlockSpec((NC, RB, D), lambda i: (0, i, 0)),
        pl.BlockSpec((RB, D), lambda i: (i, 0)),
        pl.BlockSpec((1, D), lambda i: (0, 0)),
        pl.BlockSpec((D, D), lambda i: (0, 0)),
    ],
    out_specs=pl.BlockSpec((RB, D), lambda i: (i, 0)),
    out_shape=jax.ShapeDtypeStruct((N, D), jnp.float32),
)


def _final_body(cnt_ref, a_ref, y_ref, b_ref, batch_ref, wfc_ref, bfc_ref,
                o_ref, sums, gcnt):
    i = pl.program_id(0)

    @pl.when(i == 0)
    def _():
        sums[...] = jnp.zeros_like(sums)
        gcnt[...] = jnp.zeros_like(gcnt)

    dinv = _dinv_from(cnt_ref[...])
    ab = a_ref[...]
    h = jnp.maximum((ab[0] + ab[1] + y_ref[...]) * dinv + b_ref[...], 0.0)
    gid = lax.broadcasted_iota(jnp.int32, (RB, G), 1)
    mask = (batch_ref[...] == gid).astype(jnp.float32)
    sums[...] += lax.dot_general(mask, h, (((0,), (0,)), ((), ())),
                                 preferred_element_type=jnp.float32)
    gcnt[...] += lax.dot_general(mask, jnp.ones_like(h), (((0,), (0,)), ((), ())),
                                 preferred_element_type=jnp.float32)

    @pl.when(i == NRB - 1)
    def _():
        pooled = sums[...] / jnp.maximum(gcnt[...], 1.0)
        o_ref[...] = jnp.dot(pooled, wfc_ref[...],
                             preferred_element_type=jnp.float32) + bfc_ref[...]


_final = pl.pallas_call(
    _final_body,
    grid=(NRB,),
    in_specs=[
        pl.BlockSpec((NC, RB, D), lambda i: (0, i, 0)),
        pl.BlockSpec((NC, RB, D), lambda i: (0, i, 0)),
        pl.BlockSpec((RB, D), lambda i: (i, 0)),
        pl.BlockSpec((1, D), lambda i: (0, 0)),
        pl.BlockSpec((RB, 1), lambda i: (i, 0)),
        pl.BlockSpec((D, O), lambda i: (0, 0)),
        pl.BlockSpec((1, O), lambda i: (0, 0)),
    ],
    out_specs=pl.BlockSpec((G, O), lambda i: (0, 0)),
    out_shape=jax.ShapeDtypeStruct((G, O), jnp.float32),
    scratch_shapes=[
        pltpu.VMEM((G, D), jnp.float32),
        pltpu.VMEM((G, D), jnp.float32),
    ],
)


def kernel(x, edge_index, batch, W1, b1, W2, b2, W3, b3, Wfc, bfc):
    pad = EPAD - EPW
    src = jnp.pad(edge_index[0].reshape(NW, EPW),
                  ((0, 0), (0, pad))).reshape(NW, NCHUNK, CH)
    dst = jnp.pad(edge_index[1].reshape(NW, EPW), ((0, 0), (0, pad)),
                  constant_values=N).reshape(NW, NCHUNK, CH)
    sc_count, sc_scatter = _sc_kernels()
    cnt = sc_count(dst)
    y1 = _mm1(cnt, x, W1)
    a1 = sc_scatter(y1, src, dst)
    y2 = _layer(cnt, a1, y1, b1.reshape(1, D), W2)
    a2 = sc_scatter(y2, src, dst)
    y3 = _layer(cnt, a2, y2, b2.reshape(1, D), W3)
    a3 = sc_scatter(y3, src, dst)
    return _final(cnt, a3, y3, b3.reshape(1, D), batch.reshape(N, 1),
                  Wfc, bfc.reshape(1, O))


# speedup vs baseline: 1.0010x; 1.0010x over previous
import functools

import jax
import jax.numpy as jnp
from jax import lax
from jax.experimental import pallas as pl
from jax.experimental.pallas import tpu as pltpu
from jax.experimental.pallas import tpu_sc as plsc

N = 10000
E = 320000
D = 128
G = 16
O = 64

NC, NS = 2, 16
NW = NC * NS
EPW = E // NW
CH = 80
EPAD = 10240
NCHUNK = EPAD // CH
IB = 32
NBLK = NCHUNK // IB
NPAD = 10240
RPT = NPAD // NS


def _fill(buf, rows, width, value):
    v = jnp.full((16,), value, jnp.float32)

    def row(r, carry):
        for cidx in range(width // 16):
            buf[r, pl.ds(cidx * 16, 16)] = v
        return carry

    lax.fori_loop(0, rows, row, 0)


def _zero_acc(buf, acc_sp, s):
    _fill(buf, CH, D, 0.0)
    for k in range(RPT // CH):
        pltpu.sync_copy(buf, acc_sp.at[pl.ds(s * RPT + k * CH, CH)])


def _sc_count_body(dst_hbm, out_hbm, dst_v, ones, acc_sp):
    c = lax.axis_index("c")
    s = lax.axis_index("s")
    wid = s * NC + c
    _zero_acc(ones, acc_sp, s)
    plsc.subcore_barrier()
    _fill(ones, CH, D, 1.0)

    def body(j, carry):
        pltpu.sync_copy(ones, acc_sp.at[dst_v.at[j]], add=True)
        return carry

    for blk in range(NBLK):
        pltpu.sync_copy(dst_hbm.at[wid, pl.ds(blk * IB, IB)], dst_v)
        lax.fori_loop(0, IB, body, 0)
    plsc.subcore_barrier()
    pltpu.sync_copy(acc_sp.at[pl.ds(s * RPT, RPT)],
                    out_hbm.at[c, pl.ds(s * RPT, RPT)])


def _sc_scatter_body(y_hbm, src_hbm, dst_hbm, out_hbm,
                     src_v, dst_v, buf0, buf1, acc_sp, sem0, sem1):
    c = lax.axis_index("c")
    s = lax.axis_index("s")
    wid = s * NC + c
    _zero_acc(buf0, acc_sp, s)
    plsc.subcore_barrier()

    def body(g, carry):
        j = 2 * g
        pltpu.make_async_copy(y_hbm.at[src_v.at[j]], buf0, sem0).wait()
        pltpu.sync_copy(buf0, acc_sp.at[dst_v.at[j]], add=True)
        pltpu.async_copy(y_hbm.at[src_v.at[j + 2]], buf0, sem0)
        pltpu.make_async_copy(y_hbm.at[src_v.at[j + 1]], buf1, sem1).wait()
        pltpu.sync_copy(buf1, acc_sp.at[dst_v.at[j + 1]], add=True)
        pltpu.async_copy(y_hbm.at[src_v.at[j + 3]], buf1, sem1)
        return carry

    for blk in range(NBLK):
        pltpu.sync_copy(src_hbm.at[wid, pl.ds(blk * IB, IB)], src_v)
        pltpu.sync_copy(dst_hbm.at[wid, pl.ds(blk * IB, IB)], dst_v)
        pltpu.async_copy(y_hbm.at[src_v.at[0]], buf0, sem0)
        pltpu.async_copy(y_hbm.at[src_v.at[1]], buf1, sem1)
        lax.fori_loop(0, IB // 2 - 1, body, 0)
        j = IB - 2
        pltpu.make_async_copy(y_hbm.at[src_v.at[j]], buf0, sem0).wait()
        pltpu.sync_copy(buf0, acc_sp.at[dst_v.at[j]], add=True)
        pltpu.make_async_copy(y_hbm.at[src_v.at[j + 1]], buf1, sem1).wait()
        pltpu.sync_copy(buf1, acc_sp.at[dst_v.at[j + 1]], add=True)
    plsc.subcore_barrier()
    pltpu.sync_copy(acc_sp.at[pl.ds(s * RPT, RPT)],
                    out_hbm.at[c, pl.ds(s * RPT, RPT)])


@functools.lru_cache(maxsize=None)
def _sc_kernels():
    mesh = plsc.VectorSubcoreMesh(core_axis_name="c", subcore_axis_name="s")
    count = pl.kernel(
        _sc_count_body,
        out_type=jax.ShapeDtypeStruct((NC, NPAD, D), jnp.float32),
        mesh=mesh,
        scratch_types=[
            pltpu.VMEM((IB, CH), jnp.int32),
            pltpu.VMEM((CH, D), jnp.float32),
            pltpu.VMEM_SHARED((NPAD, D), jnp.float32),
        ],
    )
    scatter = pl.kernel(
        _sc_scatter_body,
        out_type=jax.ShapeDtypeStruct((NC, NPAD, D), jnp.float32),
        mesh=mesh,
        scratch_types=[
            pltpu.VMEM((IB, CH), jnp.int32),
            pltpu.VMEM((IB, CH), jnp.int32),
            pltpu.VMEM((CH, D), jnp.float32),
            pltpu.VMEM((CH, D), jnp.float32),
            pltpu.VMEM_SHARED((NPAD, D), jnp.float32),
            pltpu.SemaphoreType.DMA,
            pltpu.SemaphoreType.DMA,
        ],
    )
    return count, scatter


RB = 1000
NRB = N // RB


def _dinv_from(cnt_blk):
    return lax.rsqrt(cnt_blk[0] + cnt_blk[1] + 1.0)


def _mm1_body(cnt_ref, x_ref, w_ref, y_ref):
    dinv = _dinv_from(cnt_ref[...])
    y_ref[...] = jnp.dot(x_ref[...], w_ref[...],
                         preferred_element_type=jnp.float32) * dinv


_mm1 = pl.pallas_call(
    _mm1_body,
    grid=(NRB,),
    in_specs=[
        pl.BlockSpec((NC, RB, D), lambda i: (0, i, 0)),
        pl.BlockSpec((RB, D), lambda i: (i, 0)),
        pl.BlockSpec((D, D), lambda i: (0, 0)),
    ],
    out_specs=pl.BlockSpec((RB, D), lambda i: (i, 0)),
    out_shape=jax.ShapeDtypeStruct((N, D), jnp.float32),
)


def _layer_body(cnt_ref, a_ref, y_ref, b_ref, w_ref, o_ref):
    dinv = _dinv_from(cnt_ref[...])
    ab = a_ref[...]
    h = jnp.maximum((ab[0] + ab[1] + y_ref[...]) * dinv + b_ref[...], 0.0)
    o_ref[...] = jnp.dot(h, w_ref[...],
                         preferred_element_type=jnp.float32) * dinv


_layer = pl.pallas_call(
    _layer_body,
    grid=(NRB,),
    in_specs=[
        pl.BlockSpec((NC, RB, D), lambda i: (0, i, 0)),
        pl.BlockSpec((NC, RB, D), lambda i: (0, i, 0)),
        pl.BlockSpec((RB, D), lambda i: (i, 0)),
        pl.BlockSpec((1, D), lambda i: (0, 0)),
        pl.BlockSpec((D, D), lambda i: (0, 0)),
    ],
    out_specs=pl.BlockSpec((RB, D), lambda i: (i, 0)),
    out_shape=jax.ShapeDtypeStruct((N, D), jnp.float32),
)


def _final_body(cnt_ref, a_ref, y_ref, b_ref, batch_ref, wfc_ref, bfc_ref,
                o_ref, sums, gcnt):
    i = pl.program_id(0)

    @pl.when(i == 0)
    def _():
        sums[...] = jnp.zeros_like(sums)
        gcnt[...] = jnp.zeros_like(gcnt)

    dinv = _dinv_from(cnt_ref[...])
    ab = a_ref[...]
    h = jnp.maximum((ab[0] + ab[1] + y_ref[...]) * dinv + b_ref[...], 0.0)
    gid = lax.broadcasted_iota(jnp.int32, (RB, G), 1)
    mask = (batch_ref[...] == gid).astype(jnp.float32)
    sums[...] += lax.dot_general(mask, h, (((0,), (0,)), ((), ())),
                                 preferred_element_type=jnp.float32)
    gcnt[...] += lax.dot_general(mask, jnp.ones_like(h), (((0,), (0,)), ((), ())),
                                 preferred_element_type=jnp.float32)

    @pl.when(i == NRB - 1)
    def _():
        pooled = sums[...] / jnp.maximum(gcnt[...], 1.0)
        o_ref[...] = jnp.dot(pooled, wfc_ref[...],
                             preferred_element_type=jnp.float32) + bfc_ref[...]


_final = pl.pallas_call(
    _final_body,
    grid=(NRB,),
    in_specs=[
        pl.BlockSpec((NC, RB, D), lambda i: (0, i, 0)),
        pl.BlockSpec((NC, RB, D), lambda i: (0, i, 0)),
        pl.BlockSpec((RB, D), lambda i: (i, 0)),
        pl.BlockSpec((1, D), lambda i: (0, 0)),
        pl.BlockSpec((RB, 1), lambda i: (i, 0)),
        pl.BlockSpec((D, O), lambda i: (0, 0)),
        pl.BlockSpec((1, O), lambda i: (0, 0)),
    ],
    out_specs=pl.BlockSpec((G, O), lambda i: (0, 0)),
    out_shape=jax.ShapeDtypeStruct((G, O), jnp.float32),
    scratch_shapes=[
        pltpu.VMEM((G, D), jnp.float32),
        pltpu.VMEM((G, D), jnp.float32),
    ],
)


def kernel(x, edge_index, batch, W1, b1, W2, b2, W3, b3, Wfc, bfc):
    pad = EPAD - EPW
    src = jnp.pad(edge_index[0].reshape(NW, EPW),
                  ((0, 0), (0, pad))).reshape(NW, NCHUNK, CH)
    dst_pad = jnp.broadcast_to(N + jnp.arange(pad, dtype=jnp.int32), (NW, pad))
    dst = jnp.concatenate([edge_index[1].reshape(NW, EPW), dst_pad],
                          axis=1).reshape(NW, NCHUNK, CH)
    sc_count, sc_scatter = _sc_kernels()
    cnt = sc_count(dst)
    y1 = _mm1(cnt, x, W1)
    a1 = sc_scatter(y1, src, dst)
    y2 = _layer(cnt, a1, y1, b1.reshape(1, D), W2)
    a2 = sc_scatter(y2, src, dst)
    y3 = _layer(cnt, a2, y2, b2.reshape(1, D), W3)
    a3 = sc_scatter(y3, src, dst)
    return _final(cnt, a3, y3, b3.reshape(1, D), batch.reshape(N, 1),
                  Wfc, bfc.reshape(1, O))


# speedup vs baseline: 1.0387x; 1.0377x over previous
import functools

import jax
import jax.numpy as jnp
from jax import lax
from jax.experimental import pallas as pl
from jax.experimental.pallas import tpu as pltpu
from jax.experimental.pallas import tpu_sc as plsc

N = 10000
E = 320000
D = 128
G = 16
O = 64

NC, NS = 2, 16
NW = NC * NS
EPW = E // NW
CH = 128
EPAD = 10240
NCHUNK = EPAD // CH
IB = 40
NBLK = NCHUNK // IB
NPAD = 10240
RPT = NPAD // NS


def _fill(buf, rows, width, value):
    v = jnp.full((16,), value, jnp.float32)

    def row(r, carry):
        for cidx in range(width // 16):
            buf[r, pl.ds(cidx * 16, 16)] = v
        return carry

    lax.fori_loop(0, rows, row, 0)


def _zero_acc(buf, acc_sp, s):
    _fill(buf, CH, D, 0.0)
    for k in range(RPT // CH):
        pltpu.sync_copy(buf, acc_sp.at[pl.ds(s * RPT + k * CH, CH)])


def _sc_count_body(dst_hbm, out_hbm, dst_v, ones, acc_sp):
    c = lax.axis_index("c")
    s = lax.axis_index("s")
    wid = s * NC + c
    _zero_acc(ones, acc_sp, s)
    plsc.subcore_barrier()
    _fill(ones, CH, D, 1.0)

    def body(j, carry):
        pltpu.sync_copy(ones, acc_sp.at[dst_v.at[j]], add=True)
        return carry

    for blk in range(NBLK):
        pltpu.sync_copy(dst_hbm.at[wid, pl.ds(blk * IB, IB)], dst_v)
        lax.fori_loop(0, IB, body, 0)
    plsc.subcore_barrier()
    pltpu.sync_copy(acc_sp.at[pl.ds(s * RPT, RPT)],
                    out_hbm.at[c, pl.ds(s * RPT, RPT)])


def _sc_scatter_body(y_hbm, src_hbm, dst_hbm, out_hbm,
                     src_v, dst_v, buf0, buf1, acc_sp, sem0, sem1):
    c = lax.axis_index("c")
    s = lax.axis_index("s")
    wid = s * NC + c
    _zero_acc(buf0, acc_sp, s)
    plsc.subcore_barrier()

    def body(g, carry):
        j = 2 * g
        pltpu.make_async_copy(y_hbm.at[src_v.at[j]], buf0, sem0).wait()
        pltpu.sync_copy(buf0, acc_sp.at[dst_v.at[j]], add=True)
        pltpu.async_copy(y_hbm.at[src_v.at[j + 2]], buf0, sem0)
        pltpu.make_async_copy(y_hbm.at[src_v.at[j + 1]], buf1, sem1).wait()
        pltpu.sync_copy(buf1, acc_sp.at[dst_v.at[j + 1]], add=True)
        pltpu.async_copy(y_hbm.at[src_v.at[j + 3]], buf1, sem1)
        return carry

    for blk in range(NBLK):
        pltpu.sync_copy(src_hbm.at[wid, pl.ds(blk * IB, IB)], src_v)
        pltpu.sync_copy(dst_hbm.at[wid, pl.ds(blk * IB, IB)], dst_v)
        pltpu.async_copy(y_hbm.at[src_v.at[0]], buf0, sem0)
        pltpu.async_copy(y_hbm.at[src_v.at[1]], buf1, sem1)
        lax.fori_loop(0, IB // 2 - 1, body, 0)
        j = IB - 2
        pltpu.make_async_copy(y_hbm.at[src_v.at[j]], buf0, sem0).wait()
        pltpu.sync_copy(buf0, acc_sp.at[dst_v.at[j]], add=True)
        pltpu.make_async_copy(y_hbm.at[src_v.at[j + 1]], buf1, sem1).wait()
        pltpu.sync_copy(buf1, acc_sp.at[dst_v.at[j + 1]], add=True)
    plsc.subcore_barrier()
    pltpu.sync_copy(acc_sp.at[pl.ds(s * RPT, RPT)],
                    out_hbm.at[c, pl.ds(s * RPT, RPT)])


@functools.lru_cache(maxsize=None)
def _sc_kernels():
    mesh = plsc.VectorSubcoreMesh(core_axis_name="c", subcore_axis_name="s")
    count = pl.kernel(
        _sc_count_body,
        out_type=jax.ShapeDtypeStruct((NC, NPAD, D), jnp.float32),
        mesh=mesh,
        scratch_types=[
            pltpu.VMEM((IB, CH), jnp.int32),
            pltpu.VMEM((CH, D), jnp.float32),
            pltpu.VMEM_SHARED((NPAD, D), jnp.float32),
        ],
    )
    scatter = pl.kernel(
        _sc_scatter_body,
        out_type=jax.ShapeDtypeStruct((NC, NPAD, D), jnp.float32),
        mesh=mesh,
        scratch_types=[
            pltpu.VMEM((IB, CH), jnp.int32),
            pltpu.VMEM((IB, CH), jnp.int32),
            pltpu.VMEM((CH, D), jnp.float32),
            pltpu.VMEM((CH, D), jnp.float32),
            pltpu.VMEM_SHARED((NPAD, D), jnp.float32),
            pltpu.SemaphoreType.DMA,
            pltpu.SemaphoreType.DMA,
        ],
    )
    return count, scatter


RB = 1000
NRB = N // RB


def _dinv_from(cnt_blk):
    return lax.rsqrt(cnt_blk[0] + cnt_blk[1] + 1.0)


def _mm1_body(cnt_ref, x_ref, w_ref, y_ref):
    dinv = _dinv_from(cnt_ref[...])
    y_ref[...] = jnp.dot(x_ref[...], w_ref[...],
                         preferred_element_type=jnp.float32) * dinv


_mm1 = pl.pallas_call(
    _mm1_body,
    grid=(NRB,),
    in_specs=[
        pl.BlockSpec((NC, RB, D), lambda i: (0, i, 0)),
        pl.BlockSpec((RB, D), lambda i: (i, 0)),
        pl.BlockSpec((D, D), lambda i: (0, 0)),
    ],
    out_specs=pl.BlockSpec((RB, D), lambda i: (i, 0)),
    out_shape=jax.ShapeDtypeStruct((N, D), jnp.float32),
)


def _layer_body(cnt_ref, a_ref, y_ref, b_ref, w_ref, o_ref):
    dinv = _dinv_from(cnt_ref[...])
    ab = a_ref[...]
    h = jnp.maximum((ab[0] + ab[1] + y_ref[...]) * dinv + b_ref[...], 0.0)
    o_ref[...] = jnp.dot(h, w_ref[...],
                         preferred_element_type=jnp.float32) * dinv


_layer = pl.pallas_call(
    _layer_body,
    grid=(NRB,),
    in_specs=[
        pl.BlockSpec((NC, RB, D), lambda i: (0, i, 0)),
        pl.BlockSpec((NC, RB, D), lambda i: (0, i, 0)),
        pl.BlockSpec((RB, D), lambda i: (i, 0)),
        pl.BlockSpec((1, D), lambda i: (0, 0)),
        pl.BlockSpec((D, D), lambda i: (0, 0)),
    ],
    out_specs=pl.BlockSpec((RB, D), lambda i: (i, 0)),
    out_shape=jax.ShapeDtypeStruct((N, D), jnp.float32),
)


def _final_body(cnt_ref, a_ref, y_ref, b_ref, batch_ref, wfc_ref, bfc_ref,
                o_ref, sums, gcnt):
    i = pl.program_id(0)

    @pl.when(i == 0)
    def _():
        sums[...] = jnp.zeros_like(sums)
        gcnt[...] = jnp.zeros_like(gcnt)

    dinv = _dinv_from(cnt_ref[...])
    ab = a_ref[...]
    h = jnp.maximum((ab[0] + ab[1] + y_ref[...]) * dinv + b_ref[...], 0.0)
    gid = lax.broadcasted_iota(jnp.int32, (RB, G), 1)
    mask = (batch_ref[...] == gid).astype(jnp.float32)
    sums[...] += lax.dot_general(mask, h, (((0,), (0,)), ((), ())),
                                 preferred_element_type=jnp.float32)
    gcnt[...] += lax.dot_general(mask, jnp.ones_like(h), (((0,), (0,)), ((), ())),
                                 preferred_element_type=jnp.float32)

    @pl.when(i == NRB - 1)
    def _():
        pooled = sums[...] / jnp.maximum(gcnt[...], 1.0)
        o_ref[...] = jnp.dot(pooled, wfc_ref[...],
                             preferred_element_type=jnp.float32) + bfc_ref[...]


_final = pl.pallas_call(
    _final_body,
    grid=(NRB,),
    in_specs=[
        pl.BlockSpec((NC, RB, D), lambda i: (0, i, 0)),
        pl.BlockSpec((NC, RB, D), lambda i: (0, i, 0)),
        pl.BlockSpec((RB, D), lambda i: (i, 0)),
        pl.BlockSpec((1, D), lambda i: (0, 0)),
        pl.BlockSpec((RB, 1), lambda i: (i, 0)),
        pl.BlockSpec((D, O), lambda i: (0, 0)),
        pl.BlockSpec((1, O), lambda i: (0, 0)),
    ],
    out_specs=pl.BlockSpec((G, O), lambda i: (0, 0)),
    out_shape=jax.ShapeDtypeStruct((G, O), jnp.float32),
    scratch_shapes=[
        pltpu.VMEM((G, D), jnp.float32),
        pltpu.VMEM((G, D), jnp.float32),
    ],
)


def kernel(x, edge_index, batch, W1, b1, W2, b2, W3, b3, Wfc, bfc):
    pad = EPAD - EPW
    src = jnp.pad(edge_index[0].reshape(NW, EPW),
                  ((0, 0), (0, pad))).reshape(NW, NCHUNK, CH)
    dst_pad = jnp.broadcast_to(N + jnp.arange(pad, dtype=jnp.int32), (NW, pad))
    dst = jnp.concatenate([edge_index[1].reshape(NW, EPW), dst_pad],
                          axis=1).reshape(NW, NCHUNK, CH)
    sc_count, sc_scatter = _sc_kernels()
    cnt = sc_count(dst)
    y1 = _mm1(cnt, x, W1)
    a1 = sc_scatter(y1, src, dst)
    y2 = _layer(cnt, a1, y1, b1.reshape(1, D), W2)
    a2 = sc_scatter(y2, src, dst)
    y3 = _layer(cnt, a2, y2, b2.reshape(1, D), W3)
    a3 = sc_scatter(y3, src, dst)
    return _final(cnt, a3, y3, b3.reshape(1, D), batch.reshape(N, 1),
                  Wfc, bfc.reshape(1, O))


# speedup vs baseline: 1.9198x; 1.8483x over previous
import functools

import jax
import jax.numpy as jnp
from jax import lax
from jax.experimental import pallas as pl
from jax.experimental.pallas import tpu as pltpu
from jax.experimental.pallas import tpu_sc as plsc

N = 10000
E = 320000
D = 128
G = 16
O = 64

NC, NS = 2, 16
NW = NC * NS
NPAD = 10240

SCH = 125
EPT = E // NS
SNCH = EPT // SCH
HALF = NPAD // NC
NDUM = 256
ACCR = HALF + NDUM
ZPT = ACCR // NS
OPT = HALF // NS
ZB = 64

CCH = 128
EPW = E // NW
EPAD = 10240
CNCH = EPAD // CCH
CIB = 40
CNBLK = CNCH // CIB
RPT = NPAD // NS


def _fill(buf, rows, width, value):
    v = jnp.full((16,), value, jnp.float32)

    def row(r, carry):
        for cidx in range(width // 16):
            buf[r, pl.ds(cidx * 16, 16)] = v
        return carry

    lax.fori_loop(0, rows, row, 0)


def _sc_count_body(dst_hbm, out_hbm, dst_v, ones, acc_sp):
    c = lax.axis_index("c")
    s = lax.axis_index("s")
    wid = s * NC + c
    _fill(ones, CCH, D, 0.0)
    for k in range(RPT // CCH):
        pltpu.sync_copy(ones, acc_sp.at[pl.ds(s * RPT + k * CCH, CCH)])
    plsc.subcore_barrier()
    _fill(ones, CCH, D, 1.0)

    def body(j, carry):
        pltpu.sync_copy(ones, acc_sp.at[dst_v.at[j]], add=True)
        return carry

    for blk in range(CNBLK):
        pltpu.sync_copy(dst_hbm.at[wid, pl.ds(blk * CIB, CIB)], dst_v)
        lax.fori_loop(0, CIB, body, 0)
    plsc.subcore_barrier()
    pltpu.sync_copy(acc_sp.at[pl.ds(s * RPT, RPT)],
                    out_hbm.at[c, pl.ds(s * RPT, RPT)])


def _sc_scatter_body(y_hbm, src_hbm, dst_hbm, out_hbm,
                     src_v, dst_v, buf0, buf1, zbuf, acc_sp, sem0, sem1):
    c = lax.axis_index("c")
    s = lax.axis_index("s")
    pltpu.sync_copy(src_hbm.at[pl.ds(s * SNCH, SNCH)], src_v)
    pltpu.sync_copy(dst_hbm.at[pl.ds(s * SNCH, SNCH)], dst_v)

    base = c * HALF

    def remap1(v):
        local = v - base
        ok = (local >= 0) & (local < HALF)
        dummy = HALF + (v & (NDUM - 1))
        return jnp.where(ok, local, dummy)

    def remap(r, carry):
        tail = remap1(dst_v[r, pl.ds(SCH - 16, 16)])
        for off in range(0, SCH - 16, 16):
            dst_v[r, pl.ds(off, 16)] = remap1(dst_v[r, pl.ds(off, 16)])
        dst_v[r, pl.ds(SCH - 16, 16)] = tail
        return carry

    lax.fori_loop(0, SNCH, remap, 0)

    _fill(zbuf, ZB, D, 0.0)
    for k in range(ZPT // ZB):
        pltpu.sync_copy(zbuf, acc_sp.at[pl.ds(s * ZPT + k * ZB, ZB)])
    _zrem = ZPT - (ZPT // ZB) * ZB
    if _zrem:
        pltpu.sync_copy(zbuf.at[pl.ds(0, _zrem)],
                        acc_sp.at[pl.ds(s * ZPT + (ZPT // ZB) * ZB, _zrem)])
    plsc.subcore_barrier()

    pltpu.async_copy(y_hbm.at[src_v.at[0]], buf0, sem0)
    pltpu.async_copy(y_hbm.at[src_v.at[1]], buf1, sem1)

    def body(g, carry):
        j = 2 * g
        pltpu.make_async_copy(y_hbm.at[src_v.at[j]], buf0, sem0).wait()
        pltpu.sync_copy(buf0, acc_sp.at[dst_v.at[j]], add=True)
        pltpu.async_copy(y_hbm.at[src_v.at[j + 2]], buf0, sem0)
        pltpu.make_async_copy(y_hbm.at[src_v.at[j + 1]], buf1, sem1).wait()
        pltpu.sync_copy(buf1, acc_sp.at[dst_v.at[j + 1]], add=True)
        pltpu.async_copy(y_hbm.at[src_v.at[j + 3]], buf1, sem1)
        return carry

    lax.fori_loop(0, SNCH // 2 - 1, body, 0)
    j = SNCH - 2
    pltpu.make_async_copy(y_hbm.at[src_v.at[j]], buf0, sem0).wait()
    pltpu.sync_copy(buf0, acc_sp.at[dst_v.at[j]], add=True)
    pltpu.make_async_copy(y_hbm.at[src_v.at[j + 1]], buf1, sem1).wait()
    pltpu.sync_copy(buf1, acc_sp.at[dst_v.at[j + 1]], add=True)
    plsc.subcore_barrier()
    pltpu.sync_copy(acc_sp.at[pl.ds(s * OPT, OPT)],
                    out_hbm.at[pl.ds(c * HALF + s * OPT, OPT)])


@functools.lru_cache(maxsize=None)
def _sc_kernels():
    mesh = plsc.VectorSubcoreMesh(core_axis_name="c", subcore_axis_name="s")
    count = pl.kernel(
        _sc_count_body,
        out_type=jax.ShapeDtypeStruct((NC, NPAD, D), jnp.float32),
        mesh=mesh,
        scratch_types=[
            pltpu.VMEM((CIB, CCH), jnp.int32),
            pltpu.VMEM((CCH, D), jnp.float32),
            pltpu.VMEM_SHARED((NPAD, D), jnp.float32),
        ],
    )
    scatter = pl.kernel(
        _sc_scatter_body,
        out_type=jax.ShapeDtypeStruct((NPAD, D), jnp.float32),
        mesh=mesh,
        scratch_types=[
            pltpu.VMEM((SNCH, SCH), jnp.int32),
            pltpu.VMEM((SNCH, SCH), jnp.int32),
            pltpu.VMEM((SCH, D), jnp.float32),
            pltpu.VMEM((SCH, D), jnp.float32),
            pltpu.VMEM((ZB, D), jnp.float32),
            pltpu.VMEM_SHARED((ACCR, D), jnp.float32),
            pltpu.SemaphoreType.DMA,
            pltpu.SemaphoreType.DMA,
        ],
    )
    return count, scatter


RB = 1000
NRB = N // RB


def _dinv_from(cnt_blk):
    return lax.rsqrt(cnt_blk[0] + cnt_blk[1] + 1.0)


def _mm1_body(cnt_ref, x_ref, w_ref, y_ref):
    dinv = _dinv_from(cnt_ref[...])
    y_ref[...] = jnp.dot(x_ref[...], w_ref[...],
                         preferred_element_type=jnp.float32) * dinv


_mm1 = pl.pallas_call(
    _mm1_body,
    grid=(NRB,),
    in_specs=[
        pl.BlockSpec((NC, RB, D), lambda i: (0, i, 0)),
        pl.BlockSpec((RB, D), lambda i: (i, 0)),
        pl.BlockSpec((D, D), lambda i: (0, 0)),
    ],
    out_specs=pl.BlockSpec((RB, D), lambda i: (i, 0)),
    out_shape=jax.ShapeDtypeStruct((N, D), jnp.float32),
)


def _layer_body(cnt_ref, a_ref, y_ref, b_ref, w_ref, o_ref):
    dinv = _dinv_from(cnt_ref[...])
    h = jnp.maximum((a_ref[...] + y_ref[...]) * dinv + b_ref[...], 0.0)
    o_ref[...] = jnp.dot(h, w_ref[...],
                         preferred_element_type=jnp.float32) * dinv


_layer = pl.pallas_call(
    _layer_body,
    grid=(NRB,),
    in_specs=[
        pl.BlockSpec((NC, RB, D), lambda i: (0, i, 0)),
        pl.BlockSpec((RB, D), lambda i: (i, 0)),
        pl.BlockSpec((RB, D), lambda i: (i, 0)),
        pl.BlockSpec((1, D), lambda i: (0, 0)),
        pl.BlockSpec((D, D), lambda i: (0, 0)),
    ],
    out_specs=pl.BlockSpec((RB, D), lambda i: (i, 0)),
    out_shape=jax.ShapeDtypeStruct((N, D), jnp.float32),
)


def _final_body(cnt_ref, a_ref, y_ref, b_ref, batch_ref, wfc_ref, bfc_ref,
                o_ref, sums, gcnt):
    i = pl.program_id(0)

    @pl.when(i == 0)
    def _():
        sums[...] = jnp.zeros_like(sums)
        gcnt[...] = jnp.zeros_like(gcnt)

    dinv = _dinv_from(cnt_ref[...])
    h = jnp.maximum((a_ref[...] + y_ref[...]) * dinv + b_ref[...], 0.0)
    gid = lax.broadcasted_iota(jnp.int32, (RB, G), 1)
    mask = (batch_ref[...] == gid).astype(jnp.float32)
    sums[...] += lax.dot_general(mask, h, (((0,), (0,)), ((), ())),
                                 preferred_element_type=jnp.float32)
    gcnt[...] += lax.dot_general(mask, jnp.ones_like(h), (((0,), (0,)), ((), ())),
                                 preferred_element_type=jnp.float32)

    @pl.when(i == NRB - 1)
    def _():
        pooled = sums[...] / jnp.maximum(gcnt[...], 1.0)
        o_ref[...] = jnp.dot(pooled, wfc_ref[...],
                             preferred_element_type=jnp.float32) + bfc_ref[...]


_final = pl.pallas_call(
    _final_body,
    grid=(NRB,),
    in_specs=[
        pl.BlockSpec((NC, RB, D), lambda i: (0, i, 0)),
        pl.BlockSpec((RB, D), lambda i: (i, 0)),
        pl.BlockSpec((RB, D), lambda i: (i, 0)),
        pl.BlockSpec((1, D), lambda i: (0, 0)),
        pl.BlockSpec((RB, 1), lambda i: (i, 0)),
        pl.BlockSpec((D, O), lambda i: (0, 0)),
        pl.BlockSpec((1, O), lambda i: (0, 0)),
    ],
    out_specs=pl.BlockSpec((G, O), lambda i: (0, 0)),
    out_shape=jax.ShapeDtypeStruct((G, O), jnp.float32),
    scratch_shapes=[
        pltpu.VMEM((G, D), jnp.float32),
        pltpu.VMEM((G, D), jnp.float32),
    ],
)


def kernel(x, edge_index, batch, W1, b1, W2, b2, W3, b3, Wfc, bfc):
    src2 = edge_index[0].reshape(E // SCH, SCH)
    dst2 = edge_index[1].reshape(E // SCH, SCH)
    pad = EPAD - EPW
    dst_pad = jnp.broadcast_to(N + jnp.arange(pad, dtype=jnp.int32), (NW, pad))
    dst3 = jnp.concatenate([edge_index[1].reshape(NW, EPW), dst_pad],
                           axis=1).reshape(NW, CNCH, CCH)
    sc_count, sc_scatter = _sc_kernels()
    cnt = sc_count(dst3)
    y1 = _mm1(cnt, x, W1)
    a1 = sc_scatter(y1, src2, dst2)
    y2 = _layer(cnt, a1, y1, b1.reshape(1, D), W2)
    a2 = sc_scatter(y2, src2, dst2)
    y3 = _layer(cnt, a2, y2, b2.reshape(1, D), W3)
    a3 = sc_scatter(y3, src2, dst2)
    return _final(cnt, a3, y3, b3.reshape(1, D), batch.reshape(N, 1),
                  Wfc, bfc.reshape(1, O))


# speedup vs baseline: 2.0273x; 1.0560x over previous
import functools

import jax
import jax.numpy as jnp
from jax import lax
from jax.experimental import pallas as pl
from jax.experimental.pallas import tpu as pltpu
from jax.experimental.pallas import tpu_sc as plsc

N = 10000
E = 320000
D = 128
G = 16
O = 64

NC, NS = 2, 16
NW = NC * NS
NPAD = 10240

SCH = 125
EPT = E // NS
SNCH = EPT // SCH
SIB = 40
SNBLK = SNCH // SIB
HALF = NPAD // NC
NDUM = 256
ACCR = HALF + NDUM
ZPT = ACCR // NS
OPT = HALF // NS
ZB = 64

CCH = 128
EPW = E // NW
EPAD = 10240
CNCH = EPAD // CCH
CIB = 40
CNBLK = CNCH // CIB
RPT = NPAD // NS


def _fill(buf, rows, width, value):
    v = jnp.full((16,), value, jnp.float32)

    def row(r, carry):
        for cidx in range(width // 16):
            buf[r, pl.ds(cidx * 16, 16)] = v
        return carry

    lax.fori_loop(0, rows, row, 0)


def _sc_count_body(dst_hbm, out_hbm, dst_v, ones, acc_sp):
    c = lax.axis_index("c")
    s = lax.axis_index("s")
    wid = s * NC + c
    _fill(ones, CCH, D, 0.0)
    for k in range(RPT // CCH):
        pltpu.sync_copy(ones, acc_sp.at[pl.ds(s * RPT + k * CCH, CCH)])
    plsc.subcore_barrier()
    _fill(ones, CCH, D, 1.0)

    def body(j, carry):
        pltpu.sync_copy(ones, acc_sp.at[dst_v.at[j]], add=True)
        return carry

    for blk in range(CNBLK):
        pltpu.sync_copy(dst_hbm.at[wid, pl.ds(blk * CIB, CIB)], dst_v)
        lax.fori_loop(0, CIB, body, 0)
    plsc.subcore_barrier()
    pltpu.sync_copy(acc_sp.at[pl.ds(s * RPT, RPT)],
                    out_hbm.at[c, pl.ds(s * RPT, RPT)])


def _sc_scatter_body(y_hbm, src_hbm, dst_hbm, out_hbm,
                     src_v, dst_v, buf0, buf1, buf2, buf3, zbuf, acc_sp,
                     sem0, sem1, sem2, sem3):
    c = lax.axis_index("c")
    s = lax.axis_index("s")
    base = c * HALF

    def remap1(v):
        local = v - base
        ok = (local >= 0) & (local < HALF)
        dummy = HALF + (v & (NDUM - 1))
        return jnp.where(ok, local, dummy)

    def remap(r, carry):
        tail = remap1(dst_v[r, pl.ds(SCH - 16, 16)])
        for off in range(0, SCH - 16, 16):
            dst_v[r, pl.ds(off, 16)] = remap1(dst_v[r, pl.ds(off, 16)])
        dst_v[r, pl.ds(SCH - 16, 16)] = tail
        return carry

    _fill(zbuf, ZB, D, 0.0)
    for k in range(ZPT // ZB):
        pltpu.sync_copy(zbuf, acc_sp.at[pl.ds(s * ZPT + k * ZB, ZB)])
    _zrem = ZPT - (ZPT // ZB) * ZB
    if _zrem:
        pltpu.sync_copy(zbuf.at[pl.ds(0, _zrem)],
                        acc_sp.at[pl.ds(s * ZPT + (ZPT // ZB) * ZB, _zrem)])
    plsc.subcore_barrier()

    bufs = (buf0, buf1, buf2, buf3)
    sems = (sem0, sem1, sem2, sem3)

    def body(g, carry):
        j = 4 * g
        for b in range(4):
            pltpu.make_async_copy(y_hbm.at[src_v.at[j + b]],
                                  bufs[b], sems[b]).wait()
            pltpu.sync_copy(bufs[b], acc_sp.at[dst_v.at[j + b]], add=True)
            pltpu.async_copy(y_hbm.at[src_v.at[j + b + 4]], bufs[b], sems[b])
        return carry

    for blk in range(SNBLK):
        pltpu.sync_copy(src_hbm.at[pl.ds((s * SNBLK + blk) * SIB, SIB)], src_v)
        pltpu.sync_copy(dst_hbm.at[pl.ds((s * SNBLK + blk) * SIB, SIB)], dst_v)
        lax.fori_loop(0, SIB, remap, 0)
        for b in range(4):
            pltpu.async_copy(y_hbm.at[src_v.at[b]], bufs[b], sems[b])
        lax.fori_loop(0, SIB // 4 - 2, body, 0)
        j = SIB - 8
        for b in range(4):
            pltpu.make_async_copy(y_hbm.at[src_v.at[j + b]],
                                  bufs[b], sems[b]).wait()
            pltpu.sync_copy(bufs[b], acc_sp.at[dst_v.at[j + b]], add=True)
            pltpu.async_copy(y_hbm.at[src_v.at[j + b + 4]], bufs[b], sems[b])
        j = SIB - 4
        for b in range(4):
            pltpu.make_async_copy(y_hbm.at[src_v.at[j + b]],
                                  bufs[b], sems[b]).wait()
            pltpu.sync_copy(bufs[b], acc_sp.at[dst_v.at[j + b]], add=True)
    plsc.subcore_barrier()
    pltpu.sync_copy(acc_sp.at[pl.ds(s * OPT, OPT)],
                    out_hbm.at[pl.ds(c * HALF + s * OPT, OPT)])


@functools.lru_cache(maxsize=None)
def _sc_kernels():
    mesh = plsc.VectorSubcoreMesh(core_axis_name="c", subcore_axis_name="s")
    count = pl.kernel(
        _sc_count_body,
        out_type=jax.ShapeDtypeStruct((NC, NPAD, D), jnp.float32),
        mesh=mesh,
        scratch_types=[
            pltpu.VMEM((CIB, CCH), jnp.int32),
            pltpu.VMEM((CCH, D), jnp.float32),
            pltpu.VMEM_SHARED((NPAD, D), jnp.float32),
        ],
    )
    scatter = pl.kernel(
        _sc_scatter_body,
        out_type=jax.ShapeDtypeStruct((NPAD, D), jnp.float32),
        mesh=mesh,
        scratch_types=[
            pltpu.VMEM((SIB, SCH), jnp.int32),
            pltpu.VMEM((SIB, SCH), jnp.int32),
            pltpu.VMEM((SCH, D), jnp.float32),
            pltpu.VMEM((SCH, D), jnp.float32),
            pltpu.VMEM((SCH, D), jnp.float32),
            pltpu.VMEM((SCH, D), jnp.float32),
            pltpu.VMEM((ZB, D), jnp.float32),
            pltpu.VMEM_SHARED((ACCR, D), jnp.float32),
            pltpu.SemaphoreType.DMA,
            pltpu.SemaphoreType.DMA,
            pltpu.SemaphoreType.DMA,
            pltpu.SemaphoreType.DMA,
        ],
    )
    return count, scatter


RB = 1000
NRB = N // RB


def _dinv_from(cnt_blk):
    return lax.rsqrt(cnt_blk[0] + cnt_blk[1] + 1.0)


def _mm1_body(cnt_ref, x_ref, w_ref, y_ref):
    dinv = _dinv_from(cnt_ref[...])
    y_ref[...] = jnp.dot(x_ref[...], w_ref[...],
                         preferred_element_type=jnp.float32) * dinv


_mm1 = pl.pallas_call(
    _mm1_body,
    grid=(NRB,),
    in_specs=[
        pl.BlockSpec((NC, RB, D), lambda i: (0, i, 0)),
        pl.BlockSpec((RB, D), lambda i: (i, 0)),
        pl.BlockSpec((D, D), lambda i: (0, 0)),
    ],
    out_specs=pl.BlockSpec((RB, D), lambda i: (i, 0)),
    out_shape=jax.ShapeDtypeStruct((N, D), jnp.float32),
)


def _layer_body(cnt_ref, a_ref, y_ref, b_ref, w_ref, o_ref):
    dinv = _dinv_from(cnt_ref[...])
    h = jnp.maximum((a_ref[...] + y_ref[...]) * dinv + b_ref[...], 0.0)
    o_ref[...] = jnp.dot(h, w_ref[...],
                         preferred_element_type=jnp.float32) * dinv


_layer = pl.pallas_call(
    _layer_body,
    grid=(NRB,),
    in_specs=[
        pl.BlockSpec((NC, RB, D), lambda i: (0, i, 0)),
        pl.BlockSpec((RB, D), lambda i: (i, 0)),
        pl.BlockSpec((RB, D), lambda i: (i, 0)),
        pl.BlockSpec((1, D), lambda i: (0, 0)),
        pl.BlockSpec((D, D), lambda i: (0, 0)),
    ],
    out_specs=pl.BlockSpec((RB, D), lambda i: (i, 0)),
    out_shape=jax.ShapeDtypeStruct((N, D), jnp.float32),
)


def _final_body(cnt_ref, a_ref, y_ref, b_ref, batch_ref, wfc_ref, bfc_ref,
                o_ref, sums, gcnt):
    i = pl.program_id(0)

    @pl.when(i == 0)
    def _():
        sums[...] = jnp.zeros_like(sums)
        gcnt[...] = jnp.zeros_like(gcnt)

    dinv = _dinv_from(cnt_ref[...])
    h = jnp.maximum((a_ref[...] + y_ref[...]) * dinv + b_ref[...], 0.0)
    gid = lax.broadcasted_iota(jnp.int32, (RB, G), 1)
    mask = (batch_ref[...] == gid).astype(jnp.float32)
    sums[...] += lax.dot_general(mask, h, (((0,), (0,)), ((), ())),
                                 preferred_element_type=jnp.float32)
    gcnt[...] += lax.dot_general(mask, jnp.ones_like(h), (((0,), (0,)), ((), ())),
                                 preferred_element_type=jnp.float32)

    @pl.when(i == NRB - 1)
    def _():
        pooled = sums[...] / jnp.maximum(gcnt[...], 1.0)
        o_ref[...] = jnp.dot(pooled, wfc_ref[...],
                             preferred_element_type=jnp.float32) + bfc_ref[...]


_final = pl.pallas_call(
    _final_body,
    grid=(NRB,),
    in_specs=[
        pl.BlockSpec((NC, RB, D), lambda i: (0, i, 0)),
        pl.BlockSpec((RB, D), lambda i: (i, 0)),
        pl.BlockSpec((RB, D), lambda i: (i, 0)),
        pl.BlockSpec((1, D), lambda i: (0, 0)),
        pl.BlockSpec((RB, 1), lambda i: (i, 0)),
        pl.BlockSpec((D, O), lambda i: (0, 0)),
        pl.BlockSpec((1, O), lambda i: (0, 0)),
    ],
    out_specs=pl.BlockSpec((G, O), lambda i: (0, 0)),
    out_shape=jax.ShapeDtypeStruct((G, O), jnp.float32),
    scratch_shapes=[
        pltpu.VMEM((G, D), jnp.float32),
        pltpu.VMEM((G, D), jnp.float32),
    ],
)


def kernel(x, edge_index, batch, W1, b1, W2, b2, W3, b3, Wfc, bfc):
    src2 = edge_index[0].reshape(E // SCH, SCH)
    dst2 = edge_index[1].reshape(E // SCH, SCH)
    pad = EPAD - EPW
    dst_pad = jnp.broadcast_to(N + jnp.arange(pad, dtype=jnp.int32), (NW, pad))
    dst3 = jnp.concatenate([edge_index[1].reshape(NW, EPW), dst_pad],
                           axis=1).reshape(NW, CNCH, CCH)
    sc_count, sc_scatter = _sc_kernels()
    cnt = sc_count(dst3)
    y1 = _mm1(cnt, x, W1)
    a1 = sc_scatter(y1, src2, dst2)
    y2 = _layer(cnt, a1, y1, b1.reshape(1, D), W2)
    a2 = sc_scatter(y2, src2, dst2)
    y3 = _layer(cnt, a2, y2, b2.reshape(1, D), W3)
    a3 = sc_scatter(y3, src2, dst2)
    return _final(cnt, a3, y3, b3.reshape(1, D), batch.reshape(N, 1),
                  Wfc, bfc.reshape(1, O))
